# Initial kernel scaffold; baseline (speedup 1.0000x reference)
import jax
import jax.numpy as jnp
from jax.experimental import pallas as pl

NMERS = 6
HID = 768
LAB = 16
B = 16
S = 512
P = S + NMERS - 1


def _kmer_kernel(emb_ref, mask_ref, wt_ref, b_ref, out_ref):
    emb = emb_ref[0]
    L = jnp.sum(mask_ref[0])
    nc = jnp.maximum(L - 2, 1)

    proj = jnp.dot(emb, wt_ref[:], preferred_element_type=jnp.float32)

    i = jax.lax.broadcasted_iota(jnp.int32, (S, LAB), 0)
    pm = jnp.where((i >= 1) & (i <= nc), proj, 0.0)

    z5 = jnp.zeros((5, LAB), dtype=jnp.float32)
    zp = jnp.concatenate([z5, pm, z5], axis=0)
    win = zp[0:P]
    for k in range(1, NMERS):
        win = win + zp[k:k + P]

    q = jax.lax.broadcasted_iota(jnp.int32, (P, LAB), 0)
    lo = jnp.maximum(1, q - 5)
    hi = jnp.minimum(nc, q)
    denom = jnp.maximum(hi - lo + 1, 1).astype(jnp.float32)
    valid = (q >= 1) & (q - 1 <= nc + 4)
    row = jnp.where(valid, win / denom, 0.0)

    row = jnp.where(q == 0, proj[0:1, :], row)
    eos_idx = jnp.where(L >= 1, L - 1, S - 1)
    eos = jax.lax.dynamic_slice(proj, (eos_idx, 0), (1, LAB))
    row = jnp.where(q == L + 4, eos, row)

    out_ref[0] = row + b_ref[:]


@jax.jit
def kernel(outputs, attention_mask, W_dec, b_dec):
    emb = outputs[0]
    mask = attention_mask.reshape(B, 1, S)
    wt = W_dec.T
    bb = b_dec.reshape(1, LAB)

    out = pl.pallas_call(
        _kmer_kernel,
        grid=(B,),
        in_specs=[
            pl.BlockSpec((1, S, HID), lambda b: (b, 0, 0)),
            pl.BlockSpec((1, 1, S), lambda b: (b, 0, 0)),
            pl.BlockSpec((HID, LAB), lambda b: (0, 0)),
            pl.BlockSpec((1, LAB), lambda b: (0, 0)),
        ],
        out_specs=pl.BlockSpec((1, P, LAB), lambda b: (b, 0, 0)),
        out_shape=jax.ShapeDtypeStruct((B, P, LAB), jnp.float32),
    )(emb, mask, wt, bb)
    return out


# speedup vs baseline: 19.8630x; 19.8630x over previous
import jax
import jax.numpy as jnp
from jax.experimental import pallas as pl

NMERS = 6
HID = 768
LAB = 16
B = 16
S = 512
P = S + NMERS - 1


def _kmer_kernel(emb_ref, mask_ref, wt_ref, b_ref, out_ref):
    emb = emb_ref[0]
    L = jnp.sum(mask_ref[0])
    nc = jnp.maximum(L - 2, 1)

    proj = jnp.dot(emb, wt_ref[:], preferred_element_type=jnp.float32)

    i = jax.lax.broadcasted_iota(jnp.int32, (S, LAB), 0)
    pm = jnp.where((i >= 1) & (i <= nc), proj, 0.0)

    z5 = jnp.zeros((5, LAB), dtype=jnp.float32)
    zp = jnp.concatenate([z5, pm, z5], axis=0)
    win = zp[0:P]
    for k in range(1, NMERS):
        win = win + zp[k:k + P]

    q = jax.lax.broadcasted_iota(jnp.int32, (P, LAB), 0)
    lo = jnp.maximum(1, q - 5)
    hi = jnp.minimum(nc, q)
    denom = jnp.maximum(hi - lo + 1, 1).astype(jnp.float32)
    valid = (q >= 1) & (q - 1 <= nc + 4)
    row = jnp.where(valid, win / denom, 0.0)

    row = jnp.where(q == 0, proj[0:1, :], row)
    eos_idx = jnp.where(L >= 1, L - 1, S - 1)
    eos = jnp.sum(jnp.where(i == eos_idx, proj, 0.0), axis=0, keepdims=True)
    row = jnp.where(q == L + 4, eos, row)

    out_ref[0] = row + b_ref[:]


@jax.jit
def kernel(outputs, attention_mask, W_dec, b_dec):
    emb = outputs[0]
    mask = attention_mask.reshape(B, 1, S)
    wt = W_dec.T
    bb = b_dec.reshape(1, LAB)

    out = pl.pallas_call(
        _kmer_kernel,
        grid=(B,),
        in_specs=[
            pl.BlockSpec((1, S, HID), lambda b: (b, 0, 0)),
            pl.BlockSpec((1, 1, S), lambda b: (b, 0, 0)),
            pl.BlockSpec((HID, LAB), lambda b: (0, 0)),
            pl.BlockSpec((1, LAB), lambda b: (0, 0)),
        ],
        out_specs=pl.BlockSpec((1, P, LAB), lambda b: (b, 0, 0)),
        out_shape=jax.ShapeDtypeStruct((B, P, LAB), jnp.float32),
    )(emb, mask, wt, bb)
    return out
